# baseline (speedup 1.0000x reference)
import jax
import jax.numpy as jnp
from jax.experimental import pallas as pl
from jax.experimental.pallas import tpu as pltpu

_BN = 1024


def _body(x_ref, logit_ref, ema_ref, out_ref):
    x = x_ref[...]
    c, bn = x.shape
    m = jnp.max(x, axis=0, keepdims=True)
    iota_f = jax.lax.broadcasted_iota(jnp.int32, (c, bn), 0).astype(jnp.float32)
    idxf = jnp.min(jnp.where(x == m, iota_f, 1024.0), axis=0, keepdims=True)
    ema_b = jnp.broadcast_to(ema_ref[...], (c, bn))
    thr = jnp.max(jnp.where(iota_f == idxf, ema_b, -jnp.inf), axis=0,
                  keepdims=True)
    d = jnp.maximum(thr[0] - logit_ref[...], 0.0)
    out_ref[...] = d * d


def kernel(gen_logits, gen_classes, ema_real):
    b, c = gen_classes.shape
    grid = b // _BN
    out = pl.pallas_call(
        _body,
        grid=(grid,),
        in_specs=[
            pl.BlockSpec((c, _BN), lambda i: (0, i)),
            pl.BlockSpec((_BN,), lambda i: (i,)),
            pl.BlockSpec((c, 1), lambda i: (0, 0)),
        ],
        out_specs=pl.BlockSpec((_BN,), lambda i: (i,)),
        out_shape=jax.ShapeDtypeStruct((b,), jnp.float32),
        compiler_params=pltpu.CompilerParams(
            dimension_semantics=("arbitrary",),
        ),
    )(gen_classes.T, gen_logits.reshape(b), ema_real.reshape(c, 1))
    return out.reshape(b, 1)


# speedup vs baseline: 1.0995x; 1.0995x over previous
import jax
import jax.numpy as jnp
from jax.experimental import pallas as pl
from jax.experimental.pallas import tpu as pltpu

_BN = 2048


def _body(x_ref, logit_ref, ema_ref, out_ref):
    x = x_ref[...]
    c, bn = x.shape
    m = jnp.max(x, axis=0, keepdims=True)
    iota_f = jax.lax.broadcasted_iota(jnp.int32, (c, bn), 0).astype(jnp.float32)
    idxf = jnp.min(jnp.where(x == m, iota_f, 1024.0), axis=0, keepdims=True)
    ema_b = jnp.broadcast_to(ema_ref[...], (c, bn))
    thr = jnp.max(jnp.where(iota_f == idxf, ema_b, -jnp.inf), axis=0,
                  keepdims=True)
    d = jnp.maximum(thr[0] - logit_ref[...], 0.0)
    out_ref[...] = d * d


def kernel(gen_logits, gen_classes, ema_real):
    b, c = gen_classes.shape
    grid = b // _BN
    out = pl.pallas_call(
        _body,
        grid=(grid,),
        in_specs=[
            pl.BlockSpec((c, _BN), lambda i: (0, i)),
            pl.BlockSpec((_BN,), lambda i: (i,)),
            pl.BlockSpec((c, 1), lambda i: (0, 0)),
        ],
        out_specs=pl.BlockSpec((_BN,), lambda i: (i,)),
        out_shape=jax.ShapeDtypeStruct((b,), jnp.float32),
        compiler_params=pltpu.CompilerParams(
            dimension_semantics=("arbitrary",),
        ),
    )(gen_classes.T, gen_logits.reshape(b), ema_real.reshape(c, 1))
    return out.reshape(b, 1)
